# baseline (speedup 1.0000x reference)
import jax
import jax.numpy as jnp
from jax import lax
from jax.experimental import pallas as pl
from jax.experimental.pallas import tpu as pltpu
from jax.experimental.pallas import tpu_sc as plsc

_B = 128
_N = 576
_K = 24
_C = 571
_NCH = _N // 16
_NW = 32
_RPW = _B // _NW


def _pre_body(x_ref, o_ref):
    o_ref[...] = x_ref[:, 1:577, 0]


def _extract_score(x):
    return pl.pallas_call(
        _pre_body,
        grid=(16,),
        in_specs=[pl.BlockSpec((8, 577, 128), lambda g: (g, 0, 0))],
        out_specs=pl.BlockSpec((8, _N), lambda g: (g, 0)),
        out_shape=jax.ShapeDtypeStruct((_B, _N), jnp.float32),
    )(x)


def _sort16(v):
    dummy = jnp.zeros((16,), jnp.int32)
    k, _ = plsc.sort_key_val(v, dummy, descending=True)
    return k


def _merge16(a, b):
    rb = lax.rev(b, (0,))
    lo = jnp.maximum(a, rb)
    hi = jnp.minimum(a, rb)
    return _sort16(lo), _sort16(hi)


def _merge32(a0, a1, b0, b1):
    m0 = jnp.maximum(a0, lax.rev(b1, (0,)))
    m1 = jnp.maximum(a1, lax.rev(b0, (0,)))
    lo = jnp.maximum(m0, m1)
    hi = jnp.minimum(m0, m1)
    return _sort16(lo), _sort16(hi)


def _top32(buf, nodes, off=0):
    @pl.loop(0, _NCH // 2)
    def _l0(i):
        a = _sort16(buf[pl.ds(off + 32 * i, 16)])
        b = _sort16(buf[pl.ds(off + 32 * i + 16, 16)])
        lo, hi = _merge16(a, b)
        nodes[pl.ds(32 * i, 16)] = lo
        nodes[pl.ds(32 * i + 16, 16)] = hi

    n = _NCH // 2
    while n > 1:
        m = n // 2

        @pl.loop(0, m)
        def _lv(i):
            a0 = nodes[pl.ds(64 * i, 16)]
            a1 = nodes[pl.ds(64 * i + 16, 16)]
            b0 = nodes[pl.ds(64 * i + 32, 16)]
            b1 = nodes[pl.ds(64 * i + 48, 16)]
            lo, hi = _merge32(a0, a1, b0, b1)
            nodes[pl.ds(32 * i, 16)] = lo
            nodes[pl.ds(32 * i + 16, 16)] = hi

        if n % 2:
            lo = nodes[pl.ds(32 * (n - 1), 16)]
            hi = nodes[pl.ds(32 * (n - 1) + 16, 16)]
            nodes[pl.ds(32 * m, 16)] = lo
            nodes[pl.ds(32 * m + 16, 16)] = hi
            n = m + 1
        else:
            n = m
    return nodes[pl.ds(0, 16)], nodes[pl.ds(16, 16)]


def _body(score_hbm, taps, idx_out, cnt_out,
          score, cntbuf, outbuf, keybuf, nodes_f, nodes_i, tscr, patch32):
    wid = lax.axis_index("c") * 16 + lax.axis_index("s")

    pltpu.sync_copy(score_hbm.at[pl.ds(wid * (_RPW * _N), _RPW * _N)], score)
    pltpu.sync_copy(taps, tscr)

    zeros16 = jnp.zeros((16,), jnp.float32)
    iota16 = lax.iota(jnp.int32, 16)

    def _bcast_lane(v, lane):
        m = jnp.max(jnp.where(iota16 == lane, v, jnp.float32(-3e38)))
        return zeros16 + m

    tv = tscr[pl.ds(0, 16)]
    tap = [_bcast_lane(tv, t) for t in range(9)]

    cntbuf[pl.ds(0, 16)] = zeros16
    cntbuf[pl.ds(16 + _N, 16)] = zeros16

    @pl.loop(0, _RPW)
    def _row(r):
        b = wid * _RPW + r
        roff = r * _N

        lo, hi = _top32(score, nodes_f, off=roff)
        tsp = _bcast_lane(hi, _K - 1 - 16)
        ngt = (plsc.all_reduce_population_count(lo > tsp)
               + plsc.all_reduce_population_count(hi > tsp))
        rem = jnp.full((16,), _K, jnp.int32) - ngt

        @pl.loop(0, _NCH, init_carry=jnp.zeros((16,), jnp.int32))
        def _sel(c, carry):
            v = score[pl.ds(roff + 16 * c, 16)]
            gt = v > tsp
            eq = v == tsp
            eqi = jnp.where(eq, jnp.int32(1), jnp.int32(0))
            pc = jnp.cumsum(eqi) + carry
            sel = gt | (eq & (pc <= rem))
            cntbuf[pl.ds(16 + 16 * c, 16)] = jnp.where(sel, jnp.float32(1.0),
                                                       jnp.float32(0.0))
            return carry + plsc.all_reduce_population_count(eq)

        @pl.loop(0, _NCH)
        def _conv(c):
            base = 15 + 16 * c
            acc = tap[0] * cntbuf[pl.ds(base, 16)]
            for t in range(1, 9):
                acc = acc + tap[t] * cntbuf[pl.ds(base + t, 16)]
            outbuf[pl.ds(roff + 16 * c, 16)] = acc
            jv = iota16 + 16 * c
            key = (acc.astype(jnp.int32) << 10) + (jnp.int32(1023) - jv)
            key = jnp.where(jv < _C, key, jnp.int32(-1))
            keybuf[pl.ds(16 * c, 16)] = key

        klo, khi = _top32(keybuf, nodes_i)
        patch32[pl.ds(r * 32, 16)] = jnp.int32(1024) - (klo & jnp.int32(1023))
        patch32[pl.ds(r * 32 + 16, 16)] = jnp.int32(1024) - (khi & jnp.int32(1023))

    pltpu.sync_copy(patch32, idx_out.at[pl.ds(wid * (_RPW * 32), _RPW * 32)])
    pltpu.sync_copy(outbuf, cnt_out.at[pl.ds(wid * (_RPW * _N), _RPW * _N)])


def kernel(x, kernel):
    mesh = plsc.VectorSubcoreMesh(core_axis_name="c", subcore_axis_name="s")
    run = pl.kernel(
        _body,
        out_type=(jax.ShapeDtypeStruct((_B * 32,), jnp.int32),
                  jax.ShapeDtypeStruct((_B * _N,), jnp.float32)),
        mesh=mesh,
        compiler_params=pltpu.CompilerParams(needs_layout_passes=False),
        scratch_types=[
            pltpu.VMEM((_RPW * _N,), jnp.float32),
            pltpu.VMEM((_N + 32,), jnp.float32),
            pltpu.VMEM((_RPW * _N,), jnp.float32),
            pltpu.VMEM((_N,), jnp.int32),
            pltpu.VMEM((_N,), jnp.float32),
            pltpu.VMEM((_N,), jnp.int32),
            pltpu.VMEM((16,), jnp.float32),
            pltpu.VMEM((_RPW * 32,), jnp.int32),
        ],
    )
    score1d = _extract_score(x).reshape(_B * _N)
    taps16 = jnp.pad(kernel.reshape(9), (0, 7))
    idx_pad, cnt_pad = run(score1d, taps16)
    return (idx_pad.reshape(_B, 32)[:, :_K],
            cnt_pad.reshape(_B, _N)[:, :_C])


# speedup vs baseline: 6.1653x; 6.1653x over previous
import jax
import jax.numpy as jnp
from jax import lax
from jax.experimental import pallas as pl
from jax.experimental.pallas import tpu as pltpu
from jax.experimental.pallas import tpu_sc as plsc

_B = 128
_N = 576
_K = 24
_C = 571
_NCH = _N // 16
_NW = 32
_RPW = _B // _NW


def _sort16(v):
    dummy = jnp.zeros((16,), jnp.int32)
    k, _ = plsc.sort_key_val(v, dummy, descending=True)
    return k


def _merge16(a, b):
    rb = lax.rev(b, (0,))
    lo = jnp.maximum(a, rb)
    hi = jnp.minimum(a, rb)
    return _sort16(lo), _sort16(hi)


def _merge32(a0, a1, b0, b1):
    m0 = jnp.maximum(a0, lax.rev(b1, (0,)))
    m1 = jnp.maximum(a1, lax.rev(b0, (0,)))
    lo = jnp.maximum(m0, m1)
    hi = jnp.minimum(m0, m1)
    return _sort16(lo), _sort16(hi)


def _top32(buf, nodes, off=0):
    @pl.loop(0, _NCH // 2)
    def _l0(i):
        a = _sort16(buf[pl.ds(off + 32 * i, 16)])
        b = _sort16(buf[pl.ds(off + 32 * i + 16, 16)])
        lo, hi = _merge16(a, b)
        nodes[pl.ds(32 * i, 16)] = lo
        nodes[pl.ds(32 * i + 16, 16)] = hi

    n = _NCH // 2
    while n > 1:
        m = n // 2

        @pl.loop(0, m)
        def _lv(i):
            a0 = nodes[pl.ds(64 * i, 16)]
            a1 = nodes[pl.ds(64 * i + 16, 16)]
            b0 = nodes[pl.ds(64 * i + 32, 16)]
            b1 = nodes[pl.ds(64 * i + 48, 16)]
            lo, hi = _merge32(a0, a1, b0, b1)
            nodes[pl.ds(32 * i, 16)] = lo
            nodes[pl.ds(32 * i + 16, 16)] = hi

        if n % 2:
            lo = nodes[pl.ds(32 * (n - 1), 16)]
            hi = nodes[pl.ds(32 * (n - 1) + 16, 16)]
            nodes[pl.ds(32 * m, 16)] = lo
            nodes[pl.ds(32 * m + 16, 16)] = hi
            n = m + 1
        else:
            n = m
    return nodes[pl.ds(0, 16)], nodes[pl.ds(16, 16)]


def _body(score_hbm, taps, idx_out, cnt_out,
          score, cntbuf, outbuf, keybuf, nodes_f, nodes_i, tscr, patch32):
    wid = lax.axis_index("c") * 16 + lax.axis_index("s")

    pltpu.sync_copy(score_hbm.at[pl.ds(wid * (_RPW * _N), _RPW * _N)], score)
    pltpu.sync_copy(taps, tscr)

    zeros16 = jnp.zeros((16,), jnp.float32)
    iota16 = lax.iota(jnp.int32, 16)

    def _bcast_lane(v, lane):
        m = jnp.max(jnp.where(iota16 == lane, v, jnp.float32(-3e38)))
        return zeros16 + m

    tv = tscr[pl.ds(0, 16)]
    tap = [_bcast_lane(tv, t) for t in range(9)]

    cntbuf[pl.ds(0, 16)] = zeros16
    cntbuf[pl.ds(16 + _N, 16)] = zeros16

    @pl.loop(0, _RPW)
    def _row(r):
        b = wid * _RPW + r
        roff = r * _N

        lo, hi = _top32(score, nodes_f, off=roff)
        tsp = _bcast_lane(hi, _K - 1 - 16)
        ngt = (plsc.all_reduce_population_count(lo > tsp)
               + plsc.all_reduce_population_count(hi > tsp))
        rem = jnp.full((16,), _K, jnp.int32) - ngt

        @pl.loop(0, _NCH, init_carry=jnp.zeros((16,), jnp.int32))
        def _sel(c, carry):
            v = score[pl.ds(roff + 16 * c, 16)]
            gt = v > tsp
            eq = v == tsp
            eqi = jnp.where(eq, jnp.int32(1), jnp.int32(0))
            pc = jnp.cumsum(eqi) + carry
            sel = gt | (eq & (pc <= rem))
            cntbuf[pl.ds(16 + 16 * c, 16)] = jnp.where(sel, jnp.float32(1.0),
                                                       jnp.float32(0.0))
            return carry + plsc.all_reduce_population_count(eq)

        @pl.loop(0, _NCH)
        def _conv(c):
            base = 15 + 16 * c
            acc = tap[0] * cntbuf[pl.ds(base, 16)]
            for t in range(1, 9):
                acc = acc + tap[t] * cntbuf[pl.ds(base + t, 16)]
            outbuf[pl.ds(roff + 16 * c, 16)] = acc
            jv = iota16 + 16 * c
            key = (acc.astype(jnp.int32) << 10) + (jnp.int32(1023) - jv)
            key = jnp.where(jv < _C, key, jnp.int32(-1))
            keybuf[pl.ds(16 * c, 16)] = key

        klo, khi = _top32(keybuf, nodes_i)
        patch32[pl.ds(r * 32, 16)] = jnp.int32(1024) - (klo & jnp.int32(1023))
        patch32[pl.ds(r * 32 + 16, 16)] = jnp.int32(1024) - (khi & jnp.int32(1023))

    pltpu.sync_copy(patch32, idx_out.at[pl.ds(wid * (_RPW * 32), _RPW * 32)])
    pltpu.sync_copy(outbuf, cnt_out.at[pl.ds(wid * (_RPW * _N), _RPW * _N)])


def kernel(x, kernel):
    mesh = plsc.VectorSubcoreMesh(core_axis_name="c", subcore_axis_name="s")
    run = pl.kernel(
        _body,
        out_type=(jax.ShapeDtypeStruct((_B * 32,), jnp.int32),
                  jax.ShapeDtypeStruct((_B * _N,), jnp.float32)),
        mesh=mesh,
        compiler_params=pltpu.CompilerParams(needs_layout_passes=False),
        scratch_types=[
            pltpu.VMEM((_RPW * _N,), jnp.float32),
            pltpu.VMEM((_N + 32,), jnp.float32),
            pltpu.VMEM((_RPW * _N,), jnp.float32),
            pltpu.VMEM((_N,), jnp.int32),
            pltpu.VMEM((_N,), jnp.float32),
            pltpu.VMEM((_N,), jnp.int32),
            pltpu.VMEM((16,), jnp.float32),
            pltpu.VMEM((_RPW * 32,), jnp.int32),
        ],
    )
    score1d = x[:, 1:, 0].reshape(_B * _N)
    taps16 = jnp.pad(kernel.reshape(9), (0, 7))
    idx_pad, cnt_pad = run(score1d, taps16)
    return (idx_pad.reshape(_B, 32)[:, :_K],
            cnt_pad.reshape(_B, _N)[:, :_C])
